# baseline (speedup 1.0000x reference)
import jax
import jax.numpy as jnp
from jax import lax
from jax.experimental import pallas as pl
from jax.experimental.pallas import tpu as pltpu
from jax.experimental.pallas import tpu_sc as plsc

_NC = 2
_NS = 16
_NW = _NC * _NS
_L = 16
_B = 16384
_EMB = 16
_PER_W = _B // _NW
_CHUNKS = _PER_W // _L
_ISEG = 128
_NSEG = _PER_W // _ISEG


def _sc_body(xu_ref, xm_ref, lr_ref, t_urc, t_mrc, t_uln, t_mln, params_ref,
             y_ref,
             idx_u, idx_m, lr_v, urc_v, mrc_v, uln_v, mln_v, w_v, out_v, sem):
  wid = lax.axis_index("s") * _NC + lax.axis_index("c")
  base = wid * _PER_W

  pltpu.sync_copy(xu_ref.at[pl.ds(wid * _NSEG, _NSEG)], idx_u)
  pltpu.sync_copy(xm_ref.at[pl.ds(wid * _NSEG, _NSEG)], idx_m)
  pltpu.sync_copy(lr_ref.at[pl.ds(base, _PER_W)], lr_v)
  pltpu.sync_copy(params_ref, w_v)

  copies = []
  for k in range(_NSEG):
    seg = pl.ds(k * _ISEG, _ISEG)
    copies.append(pltpu.async_copy(t_urc.at[idx_u.at[k]], urc_v.at[seg], sem))
    copies.append(pltpu.async_copy(t_mrc.at[idx_m.at[k]], mrc_v.at[seg], sem))
    copies.append(pltpu.async_copy(t_uln.at[idx_u.at[k]], uln_v.at[seg], sem))
    copies.append(pltpu.async_copy(t_mln.at[idx_m.at[k]], mln_v.at[seg], sem))

  def _exp(x):
    t = x * 1.4426950408889634
    k = (t + 12582912.0) - 12582912.0
    y = (t - k) * 0.6931471805599453
    p = 1.0 + y * (1.0 + y * (0.5 + y * (
        0.16666666666666666 + y * (0.041666666666666664 + y * (
            0.008333333333333333 + y * 0.001388888888888889)))))
    ki = k.astype(jnp.int32)
    return p * plsc.bitcast((ki + 127) << 23, jnp.float32)

  w_v[pl.ds(_EMB, _EMB)] = _exp(w_v[pl.ds(_EMB, _EMB)])
  w_v[pl.ds(2 * _EMB, _EMB)] = _exp(w_v[pl.ds(2 * _EMB, _EMB)])
  plsc.subcore_barrier()

  iota = lax.iota(jnp.int32, _L)
  sw_rc = [plsc.load_gather(w_v, [jnp.full((_L,), _EMB + j, jnp.int32)])
           for j in range(_EMB)]
  sw_ln = [plsc.load_gather(w_v, [jnp.full((_L,), 2 * _EMB + j, jnp.int32)])
           for j in range(_EMB)]
  s_rc_vec = sw_rc[0]
  s_ln_vec = sw_ln[0]
  for j in range(1, _EMB):
    s_rc_vec = s_rc_vec + sw_rc[j]
    s_ln_vec = s_ln_vec + sw_ln[j]
  scale_rc = 0.5 / s_rc_vec
  scale_ln = 0.5 / s_ln_vec
  sw_rc = [v * scale_rc for v in sw_rc]
  sw_ln = [v * scale_ln for v in sw_ln]
  b_rc = plsc.load_gather(w_v, [jnp.full((_L,), 3 * _EMB, jnp.int32)])
  b_ln = plsc.load_gather(w_v, [jnp.full((_L,), 3 * _EMB + 1, jnp.int32)])

  for c in copies:
    c.wait()
  plsc.subcore_barrier()

  def chunk(r, carry):
    row0 = r * _L
    rows = row0 + iota

    def head(u_ref, m_ref, sw):
      acc = jnp.zeros((_L,), jnp.float32)
      for j in range(_EMB):
        col = jnp.full((_L,), j, jnp.int32)
        ucol = plsc.load_gather(u_ref, [rows, col])
        mcol = plsc.load_gather(m_ref, [rows, col])
        acc = acc + (ucol - mcol) * sw[j]
      return acc

    x_rc = head(urc_v, mrc_v, sw_rc) + b_rc
    x_ln = head(uln_v, mln_v, sw_ln) + b_ln
    s_rc = 1.0 / (1.0 + _exp(-x_rc))
    s_ln = 1.0 / (1.0 + _exp(-x_ln))
    lr = lr_v[pl.ds(row0, _L)]
    out_v[pl.ds(row0, _L)] = s_rc * lr + s_ln * (1.0 - lr)
    return carry

  lax.fori_loop(0, _CHUNKS, chunk, 0)
  pltpu.sync_copy(out_v, y_ref.at[pl.ds(base, _PER_W)])


def kernel(x_uid, x_mid, ln_ratio, uid_emb_rc, mid_emb_rc, uid_emb_ln,
           mid_emb_ln, w_rc, b_rc, w_ln, b_ln):
  params = jnp.concatenate(
      [jnp.zeros((16,), jnp.float32), w_rc, w_ln, b_rc, b_ln,
       jnp.zeros((78,), jnp.float32)])
  xu = x_uid.astype(jnp.int32).reshape(_B // _ISEG, _ISEG)
  xm = x_mid.astype(jnp.int32).reshape(_B // _ISEG, _ISEG)

  mesh = plsc.VectorSubcoreMesh(core_axis_name="c", subcore_axis_name="s")
  f = pl.kernel(
      _sc_body,
      out_type=jax.ShapeDtypeStruct((_B,), jnp.float32),
      mesh=mesh,
      compiler_params=pltpu.CompilerParams(
          needs_layout_passes=False, use_tc_tiling_on_sc=False),
      scratch_types=[
          pltpu.VMEM((_NSEG, _ISEG), jnp.int32),
          pltpu.VMEM((_NSEG, _ISEG), jnp.int32),
          pltpu.VMEM((_PER_W,), jnp.float32),
          pltpu.VMEM((_PER_W, _EMB), jnp.float32),
          pltpu.VMEM((_PER_W, _EMB), jnp.float32),
          pltpu.VMEM((_PER_W, _EMB), jnp.float32),
          pltpu.VMEM((_PER_W, _EMB), jnp.float32),
          pltpu.VMEM((128,), jnp.float32),
          pltpu.VMEM((_PER_W,), jnp.float32),
          pltpu.SemaphoreType.DMA,
      ],
  )
  return f(xu, xm, ln_ratio, uid_emb_rc, mid_emb_rc, uid_emb_ln, mid_emb_ln,
           params)


# speedup vs baseline: 1.3983x; 1.3983x over previous
import jax
import jax.numpy as jnp
from jax import lax
from jax.experimental import pallas as pl
from jax.experimental.pallas import tpu as pltpu
from jax.experimental.pallas import tpu_sc as plsc

_NC = 2
_NS = 16
_NW = _NC * _NS
_L = 16
_B = 16384
_EMB = 16
_PER_W = _B // _NW
_CHUNKS = _PER_W // _L
_PAIRS = _CHUNKS // 2


def _sc_body(xu_ref, xm_ref, lr_ref, t_urc, t_mrc, t_uln, t_mln, params_ref,
             y_ref,
             idx_u, idx_m, lr_v,
             a_urc, a_mrc, a_uln, a_mln,
             b_urc, b_mrc, b_uln, b_mln,
             w_v, out_v, sem_a, sem_b):
  wid = lax.axis_index("s") * _NC + lax.axis_index("c")
  base = wid * _PER_W
  tabs = (t_urc, t_mrc, t_uln, t_mln)
  bufs_a = (a_urc, a_mrc, a_uln, a_mln)
  bufs_b = (b_urc, b_mrc, b_uln, b_mln)

  pltpu.sync_copy(xu_ref.at[pl.ds(base, _PER_W)], idx_u)
  pltpu.sync_copy(xm_ref.at[pl.ds(base, _PER_W)], idx_m)
  pltpu.sync_copy(lr_ref.at[pl.ds(base, _PER_W)], lr_v)
  pltpu.sync_copy(params_ref, w_v)

  def enq(row0, bufs, sem):
    iu = idx_u[pl.ds(row0, _L)]
    im = idx_m[pl.ds(row0, _L)]
    for r in range(_L):
      iur = iu[r]
      imr = im[r]
      pltpu.async_copy(t_urc.at[pl.ds(iur, 1)], bufs[0].at[pl.ds(r, 1)], sem)
      pltpu.async_copy(t_mrc.at[pl.ds(imr, 1)], bufs[1].at[pl.ds(r, 1)], sem)
      pltpu.async_copy(t_uln.at[pl.ds(iur, 1)], bufs[2].at[pl.ds(r, 1)], sem)
      pltpu.async_copy(t_mln.at[pl.ds(imr, 1)], bufs[3].at[pl.ds(r, 1)], sem)

  def drain(bufs, sem):
    for r in range(_L):
      for t, b in zip(tabs, bufs):
        pltpu.make_async_copy(
            t.at[pl.ds(0, 1)], b.at[pl.ds(r, 1)], sem).wait()

  enq(0, bufs_a, sem_a)

  def _exp(x):
    t = x * 1.4426950408889634
    k = (t + 12582912.0) - 12582912.0
    y = (t - k) * 0.6931471805599453
    p = 1.0 + y * (1.0 + y * (0.5 + y * (
        0.16666666666666666 + y * (0.041666666666666664 + y * (
            0.008333333333333333 + y * 0.001388888888888889)))))
    ki = k.astype(jnp.int32)
    return p * plsc.bitcast((ki + 127) << 23, jnp.float32)

  w_v[pl.ds(_EMB, _EMB)] = _exp(w_v[pl.ds(_EMB, _EMB)])
  w_v[pl.ds(2 * _EMB, _EMB)] = _exp(w_v[pl.ds(2 * _EMB, _EMB)])
  plsc.subcore_barrier()

  iota = lax.iota(jnp.int32, _L)
  sw_rc = [plsc.load_gather(w_v, [jnp.full((_L,), _EMB + j, jnp.int32)])
           for j in range(_EMB)]
  sw_ln = [plsc.load_gather(w_v, [jnp.full((_L,), 2 * _EMB + j, jnp.int32)])
           for j in range(_EMB)]
  s_rc_vec = sw_rc[0]
  s_ln_vec = sw_ln[0]
  for j in range(1, _EMB):
    s_rc_vec = s_rc_vec + sw_rc[j]
    s_ln_vec = s_ln_vec + sw_ln[j]
  scale_rc = 0.5 / s_rc_vec
  scale_ln = 0.5 / s_ln_vec
  sw_rc = [v * scale_rc for v in sw_rc]
  sw_ln = [v * scale_ln for v in sw_ln]
  b_rc = plsc.load_gather(w_v, [jnp.full((_L,), 3 * _EMB, jnp.int32)])
  b_ln = plsc.load_gather(w_v, [jnp.full((_L,), 3 * _EMB + 1, jnp.int32)])

  def compute(row0, bufs):
    def head(u_ref, m_ref, sw):
      acc = jnp.zeros((_L,), jnp.float32)
      for j in range(_EMB):
        col = jnp.full((_L,), j, jnp.int32)
        ucol = plsc.load_gather(u_ref, [iota, col])
        mcol = plsc.load_gather(m_ref, [iota, col])
        acc = acc + (ucol - mcol) * sw[j]
      return acc

    x_rc = head(bufs[0], bufs[1], sw_rc) + b_rc
    x_ln = head(bufs[2], bufs[3], sw_ln) + b_ln
    s_rc = 1.0 / (1.0 + _exp(-x_rc))
    s_ln = 1.0 / (1.0 + _exp(-x_ln))
    lr = lr_v[pl.ds(row0, _L)]
    out_v[pl.ds(row0, _L)] = s_rc * lr + s_ln * (1.0 - lr)

  def pair_body(p, carry):
    row_a = (2 * p) * _L
    row_b = row_a + _L
    enq(row_b, bufs_b, sem_b)
    drain(bufs_a, sem_a)
    compute(row_a, bufs_a)

    @pl.when(p + 1 < _PAIRS)
    def _():
      enq(row_b + _L, bufs_a, sem_a)

    drain(bufs_b, sem_b)
    compute(row_b, bufs_b)
    return carry

  lax.fori_loop(0, _PAIRS, pair_body, 0)
  pltpu.sync_copy(out_v, y_ref.at[pl.ds(base, _PER_W)])


def kernel(x_uid, x_mid, ln_ratio, uid_emb_rc, mid_emb_rc, uid_emb_ln,
           mid_emb_ln, w_rc, b_rc, w_ln, b_ln):
  params = jnp.concatenate(
      [jnp.zeros((16,), jnp.float32), w_rc, w_ln, b_rc, b_ln,
       jnp.zeros((78,), jnp.float32)])
  xu = x_uid.astype(jnp.int32)
  xm = x_mid.astype(jnp.int32)

  mesh = plsc.VectorSubcoreMesh(core_axis_name="c", subcore_axis_name="s")
  row_buf = pltpu.VMEM((_L, _EMB), jnp.float32)
  f = pl.kernel(
      _sc_body,
      out_type=jax.ShapeDtypeStruct((_B,), jnp.float32),
      mesh=mesh,
      compiler_params=pltpu.CompilerParams(needs_layout_passes=False),
      scratch_types=[
          pltpu.VMEM((_PER_W,), jnp.int32),
          pltpu.VMEM((_PER_W,), jnp.int32),
          pltpu.VMEM((_PER_W,), jnp.float32),
          row_buf, row_buf, row_buf, row_buf,
          row_buf, row_buf, row_buf, row_buf,
          pltpu.VMEM((128,), jnp.float32),
          pltpu.VMEM((_PER_W,), jnp.float32),
          pltpu.SemaphoreType.DMA,
          pltpu.SemaphoreType.DMA,
      ],
  )
  return f(xu, xm, ln_ratio, uid_emb_rc, mid_emb_rc, uid_emb_ln, mid_emb_ln,
           params)
